# baseline (speedup 1.0000x reference)
import functools

import jax
import jax.numpy as jnp
from jax import lax
from jax.experimental import pallas as pl
from jax.experimental.pallas import tpu as pltpu
from jax.experimental.pallas import tpu_sc as plsc

_N = 10000
_NP = 10240
_F = 128
_E = 320000
_NC = 2
_NS = 16
_NW = _NC * _NS
_CH = 128
_G = 81
_GB = 27
_EALL = _NW * _G * _CH
_GB2 = 24
_NBLK = 7
_NBLK_SLOW = 3
_SLOW_C = 0
_EALL2 = _NS * _NBLK * _GB2 * _CH
_RPT = _NP // _NS
_BN = 512
_GRID = _NP // _BN

_f32 = jnp.float32


def _make_sc_scatter(depth):
  mesh = plsc.VectorSubcoreMesh(core_axis_name="c", subcore_axis_name="s")

  @functools.partial(
      pl.kernel,
      mesh=mesh,
      out_type=jax.ShapeDtypeStruct((_NC, _NP, depth), _f32),
      scratch_types=[
          pltpu.VMEM((_GB2, _CH), jnp.int32),
          pltpu.VMEM((_GB2, _CH), jnp.int32),
          pltpu.VMEM((2, _CH, depth), _f32),
          pltpu.VMEM_SHARED((_NP, depth), _f32),
          pltpu.SemaphoreType.DMA,
          pltpu.SemaphoreType.DMA,
      ],
  )
  def sc_scatter(y_hbm, src_hbm, dst_hbm, zeros_hbm, out_hbm,
                 idx_s, idx_d, rows, acc, sem, sem_s):
    c = lax.axis_index("c")
    s = lax.axis_index("s")
    pltpu.sync_copy(zeros_hbm, acc.at[pl.ds(s * _RPT, _RPT)])
    plsc.subcore_barrier()

    def run_blocks(b0, nb):
      def block(bi, carry):
        pltpu.sync_copy(src_hbm.at[s, b0 + bi], idx_s)
        pltpu.sync_copy(dst_hbm.at[s, b0 + bi], idx_d)
        pltpu.async_copy(y_hbm.at[idx_s.at[0]], rows.at[0], sem)

        def body(j, carry2):
          pltpu.make_async_copy(y_hbm.at[pl.ds(0, _CH)], rows.at[j % 2],
                                sem).wait()

          @pl.when(j >= 1)
          def _():
            pltpu.make_async_copy(y_hbm.at[pl.ds(0, _CH)],
                                  rows.at[(j + 1) % 2], sem_s).wait()

          @pl.when(j + 1 < _GB2)
          def _():
            pltpu.async_copy(y_hbm.at[idx_s.at[j + 1]], rows.at[(j + 1) % 2],
                             sem)

          pltpu.async_copy(rows.at[j % 2], acc.at[idx_d.at[j]], sem_s,
                           add=True)
          return carry2

        r = lax.fori_loop(0, _GB2, body, carry)
        pltpu.make_async_copy(y_hbm.at[pl.ds(0, _CH)], rows.at[0],
                              sem_s).wait()
        return r

      lax.fori_loop(0, nb, block, 0)

    @pl.when(c == _SLOW_C)
    def _():
      run_blocks(0, _NBLK_SLOW)

    @pl.when(c != _SLOW_C)
    def _():
      run_blocks(_NBLK_SLOW, _NBLK - _NBLK_SLOW)

    plsc.subcore_barrier()
    pltpu.sync_copy(acc.at[pl.ds(s * _RPT, _RPT)],
                    out_hbm.at[c, pl.ds(s * _RPT, _RPT)])

  return sc_scatter


_sc_scatter128 = _make_sc_scatter(_F)


def _make_sc_degree():
  mesh = plsc.VectorSubcoreMesh(core_axis_name="c", subcore_axis_name="s")

  @functools.partial(
      pl.kernel,
      mesh=mesh,
      out_type=jax.ShapeDtypeStruct((_NC, _NP, _F), _f32),
      scratch_types=[
          pltpu.VMEM((_GB, _CH), jnp.int32),
          pltpu.VMEM((_CH, _F), _f32),
          pltpu.VMEM_SHARED((_NP, _F), _f32),
          pltpu.SemaphoreType.DMA,
      ],
  )
  def sc_degree(ones_hbm, dst_hbm, zeros_hbm, out_hbm, idx_d, rows, acc, sem):
    c = lax.axis_index("c")
    s = lax.axis_index("s")
    wid = s * _NC + c
    pltpu.sync_copy(zeros_hbm, acc.at[pl.ds(s * _RPT, _RPT)])
    pltpu.sync_copy(ones_hbm, rows)
    plsc.subcore_barrier()

    def block(bi, carry):
      pltpu.sync_copy(dst_hbm.at[wid, bi], idx_d)

      def body(j, carry2):
        pltpu.sync_copy(rows, acc.at[idx_d.at[j]], add=True)
        return carry2

      return lax.fori_loop(0, _GB, body, carry)

    lax.fori_loop(0, _G // _GB, block, 0)
    plsc.subcore_barrier()
    pltpu.sync_copy(acc.at[pl.ds(s * _RPT, _RPT)],
                    out_hbm.at[c, pl.ds(s * _RPT, _RPT)])

  return sc_degree


_sc_degree = _make_sc_degree()


def _tc0_body(x_ref, degp_ref, mask_ref, w_ref, y_ref, dis_ref):
  deg = degp_ref[0] + degp_ref[1]
  dis = mask_ref[...] * lax.rsqrt(jnp.maximum(deg, 1.0))
  dis_ref[...] = dis
  y_ref[...] = dis * jnp.dot(x_ref[...], w_ref[...], preferred_element_type=_f32)


def _tc0(x_pad, degp, mask128, w0):
  return pl.pallas_call(
      _tc0_body,
      grid=(_GRID,),
      in_specs=[
          pl.BlockSpec((_BN, _F), lambda i: (i, 0)),
          pl.BlockSpec((_NC, _BN, _F), lambda i: (0, i, 0)),
          pl.BlockSpec((_BN, _F), lambda i: (i, 0)),
          pl.BlockSpec((_F, _F), lambda i: (0, 0)),
      ],
      out_specs=[
          pl.BlockSpec((_BN, _F), lambda i: (i, 0)),
          pl.BlockSpec((_BN, _F), lambda i: (i, 0)),
      ],
      out_shape=[
          jax.ShapeDtypeStruct((_NP, _F), _f32),
          jax.ShapeDtypeStruct((_NP, _F), _f32),
      ],
  )(x_pad, degp, mask128, w0)


def _tc_layer_body(dout, p_ref, yin_ref, dis_ref, b_ref, w_ref, out_ref, y_ref):
  dis = dis_ref[...]
  out = dis * (p_ref[0] + p_ref[1] + yin_ref[...]) + b_ref[...]
  out_ref[...] = out
  yw = jnp.dot(out, w_ref[...], preferred_element_type=_f32)
  y_ref[...] = dis[:, :dout] * yw


def _tc_layer(p, yin, disf, b_row, w, dout):
  return pl.pallas_call(
      functools.partial(_tc_layer_body, dout),
      grid=(_GRID,),
      in_specs=[
          pl.BlockSpec((_NC, _BN, _F), lambda i: (0, i, 0)),
          pl.BlockSpec((_BN, _F), lambda i: (i, 0)),
          pl.BlockSpec((_BN, _F), lambda i: (i, 0)),
          pl.BlockSpec((1, _F), lambda i: (0, 0)),
          pl.BlockSpec((_F, dout), lambda i: (0, 0)),
      ],
      out_specs=[
          pl.BlockSpec((_BN, _F), lambda i: (i, 0)),
          pl.BlockSpec((_BN, dout), lambda i: (i, 0)),
      ],
      out_shape=[
          jax.ShapeDtypeStruct((_NP, _F), _f32),
          jax.ShapeDtypeStruct((_NP, dout), _f32),
      ],
  )(p, yin, disf, b_row, w)


def _tc_final_body(p3_ref, y3_ref, dis_ref, b3_ref, o1_ref, o2_ref, o3_ref,
                   wa_ref, wb_ref, wc_ref, w4_ref, e_ref, bc_ref, res_ref):
  s3 = (p3_ref[0] + p3_ref[1] + y3_ref[...])[:, :16]
  out4_16 = dis_ref[...][:, :16] * s3 + b3_ref[...]
  out4_rep = jnp.dot(out4_16, e_ref[...], preferred_element_type=_f32)
  res = jnp.dot(o1_ref[...], wa_ref[...], preferred_element_type=_f32)
  res += jnp.dot(o2_ref[...], wb_ref[...], preferred_element_type=_f32)
  res += jnp.dot(o3_ref[...], wc_ref[...], preferred_element_type=_f32)
  res_ref[...] = res + out4_rep * w4_ref[...] + bc_ref[...]


def _tc_final(p3, y3, disf, b3_16, o1, o2, o3, wa, wb, wc, w4, e16x16, bc_row):
  return pl.pallas_call(
      _tc_final_body,
      grid=(_GRID,),
      in_specs=[
          pl.BlockSpec((_NC, _BN, _F), lambda i: (0, i, 0)),
          pl.BlockSpec((_BN, _F), lambda i: (i, 0)),
          pl.BlockSpec((_BN, _F), lambda i: (i, 0)),
          pl.BlockSpec((1, 16), lambda i: (0, 0)),
          pl.BlockSpec((_BN, _F), lambda i: (i, 0)),
          pl.BlockSpec((_BN, _F), lambda i: (i, 0)),
          pl.BlockSpec((_BN, _F), lambda i: (i, 0)),
          pl.BlockSpec((_F, 16), lambda i: (0, 0)),
          pl.BlockSpec((_F, 16), lambda i: (0, 0)),
          pl.BlockSpec((_F, 16), lambda i: (0, 0)),
          pl.BlockSpec((1, 16), lambda i: (0, 0)),
          pl.BlockSpec((16, 16), lambda i: (0, 0)),
          pl.BlockSpec((1, 16), lambda i: (0, 0)),
      ],
      out_specs=pl.BlockSpec((_BN, 16), lambda i: (i, 0)),
      out_shape=jax.ShapeDtypeStruct((_NP, 16), _f32),
  )(p3, y3, disf, b3_16, o1, o2, o3, wa, wb, wc, w4, e16x16, bc_row)


def kernel(x, edge_index, edge_attr, W0, b0, W1, b1, W2, b2, W3, b3, Wc, bc):
  src = edge_index[0].astype(jnp.int32)
  dst = edge_index[1].astype(jnp.int32)
  loops = jnp.arange(_N, dtype=jnp.int32)
  fill = jnp.full((_EALL - _E - _N,), _NP - 1, dtype=jnp.int32)
  dst_r = jnp.concatenate([dst, loops, fill]).reshape(_NW, _G // _GB, _GB, _CH)
  fill2 = _N + (jnp.arange(_EALL2 - _E, dtype=jnp.int32) % (_NP - _N))
  src_r2 = jnp.concatenate([src, fill2]).reshape(_NS, _NBLK, _GB2, _CH)
  dst_r2 = jnp.concatenate([dst, fill2]).reshape(_NS, _NBLK, _GB2, _CH)

  x_pad = jnp.pad(x, ((0, _NP - _N), (0, 0)))
  zeros128 = jnp.zeros((_RPT, _F), _f32)
  ones128 = jnp.ones((_CH, _F), _f32)
  mask128 = jnp.broadcast_to(
      (jnp.arange(_NP) < _N).astype(_f32)[:, None], (_NP, _F))
  e16x16 = jnp.zeros((16, 16), _f32).at[0].set(1.0)

  w3p = jnp.pad(W3, ((0, 0), (0, _F - 1)))
  wc_flat = Wc[:, 0, :]
  wa = wc_flat[:, 0:128].T
  wb = wc_flat[:, 128:256].T
  wcc = wc_flat[:, 256:384].T
  w4 = wc_flat[:, 384][None, :]
  b3_16 = jnp.broadcast_to(b3, (1, 16))
  bc_row = bc[None, :]

  degp = _sc_degree(ones128, dst_r, zeros128)
  y0, disf = _tc0(x_pad, degp, mask128, W0)
  p0 = _sc_scatter128(y0, src_r2, dst_r2, zeros128)
  out1, y1 = _tc_layer(p0, y0, disf, b0[None, :], W1, _F)
  p1 = _sc_scatter128(y1, src_r2, dst_r2, zeros128)
  out2, y2 = _tc_layer(p1, y1, disf, b1[None, :], W2, _F)
  p2 = _sc_scatter128(y2, src_r2, dst_r2, zeros128)
  out3, y3 = _tc_layer(p2, y2, disf, b2[None, :], w3p, _F)
  p3 = _sc_scatter128(y3, src_r2, dst_r2, zeros128)
  res = _tc_final(p3, y3, disf, b3_16, out1, out2, out3, wa, wb, wcc, w4,
                  e16x16, bc_row)
  return res[:_N]


# speedup vs baseline: 1.0849x; 1.0849x over previous
import functools

import jax
import jax.numpy as jnp
from jax import lax
from jax.experimental import pallas as pl
from jax.experimental.pallas import tpu as pltpu
from jax.experimental.pallas import tpu_sc as plsc

_N = 10000
_NP = 10240
_F = 128
_E = 320000
_NC = 2
_NS = 16
_NW = _NC * _NS
_CH = 128
_G = 81
_GB = 27
_EALL = _NW * _G * _CH
_G2 = 84
_GB2 = 28
_EALL2 = _NW * _G2 * _CH
_RPT = _NP // _NS
_BN = 512
_GRID = _NP // _BN

_f32 = jnp.float32


def _make_sc_scatter(depth):
  mesh = plsc.VectorSubcoreMesh(core_axis_name="c", subcore_axis_name="s")

  @functools.partial(
      pl.kernel,
      mesh=mesh,
      out_type=jax.ShapeDtypeStruct((_NC, _NP, depth), _f32),
      scratch_types=[
          pltpu.VMEM((_GB2, _CH), jnp.int32),
          pltpu.VMEM((_GB2, _CH), jnp.int32),
          pltpu.VMEM((2, _CH, depth), _f32),
          pltpu.VMEM_SHARED((_NP, depth), _f32),
          pltpu.SemaphoreType.DMA,
          pltpu.SemaphoreType.DMA,
      ],
  )
  def sc_scatter(y_hbm, src_hbm, dst_hbm, zeros_hbm, out_hbm,
                 idx_s, idx_d, rows, acc, sem, sem_s):
    c = lax.axis_index("c")
    s = lax.axis_index("s")
    pltpu.sync_copy(zeros_hbm, acc.at[pl.ds(s * _RPT, _RPT)])
    plsc.subcore_barrier()

    wid = s * _NC + c

    def run_blocks(b0, nb):
      def block(bi, carry):
        pltpu.sync_copy(src_hbm.at[wid, b0 + bi], idx_s)
        pltpu.sync_copy(dst_hbm.at[wid, b0 + bi], idx_d)
        pltpu.async_copy(y_hbm.at[idx_s.at[0]], rows.at[0], sem)

        def body(j, carry2):
          pltpu.make_async_copy(y_hbm.at[pl.ds(0, _CH)], rows.at[j % 2],
                                sem).wait()

          @pl.when(j >= 1)
          def _():
            pltpu.make_async_copy(y_hbm.at[pl.ds(0, _CH)],
                                  rows.at[(j + 1) % 2], sem_s).wait()

          @pl.when(j + 1 < _GB2)
          def _():
            pltpu.async_copy(y_hbm.at[idx_s.at[j + 1]], rows.at[(j + 1) % 2],
                             sem)

          pltpu.async_copy(rows.at[j % 2], acc.at[idx_d.at[j]], sem_s,
                           add=True)
          return carry2

        r = lax.fori_loop(0, _GB2, body, carry)
        pltpu.make_async_copy(y_hbm.at[pl.ds(0, _CH)], rows.at[0],
                              sem_s).wait()
        return r

      lax.fori_loop(0, nb, block, 0)

    run_blocks(0, _G2 // _GB2)
    plsc.subcore_barrier()
    pltpu.sync_copy(acc.at[pl.ds(s * _RPT, _RPT)],
                    out_hbm.at[c, pl.ds(s * _RPT, _RPT)])

  return sc_scatter


_sc_scatter128 = _make_sc_scatter(_F)


def _make_sc_degree():
  mesh = plsc.VectorSubcoreMesh(core_axis_name="c", subcore_axis_name="s")

  @functools.partial(
      pl.kernel,
      mesh=mesh,
      out_type=jax.ShapeDtypeStruct((_NC, _NP, _F), _f32),
      scratch_types=[
          pltpu.VMEM((_GB, _CH), jnp.int32),
          pltpu.VMEM((_CH, _F), _f32),
          pltpu.VMEM_SHARED((_NP, _F), _f32),
          pltpu.SemaphoreType.DMA,
      ],
  )
  def sc_degree(ones_hbm, dst_hbm, zeros_hbm, out_hbm, idx_d, rows, acc, sem):
    c = lax.axis_index("c")
    s = lax.axis_index("s")
    wid = s * _NC + c
    pltpu.sync_copy(zeros_hbm, acc.at[pl.ds(s * _RPT, _RPT)])
    pltpu.sync_copy(ones_hbm, rows)
    plsc.subcore_barrier()

    def block(bi, carry):
      pltpu.sync_copy(dst_hbm.at[wid, bi], idx_d)

      def body(j, carry2):
        pltpu.sync_copy(rows, acc.at[idx_d.at[j]], add=True)
        return carry2

      return lax.fori_loop(0, _GB, body, carry)

    lax.fori_loop(0, _G // _GB, block, 0)
    plsc.subcore_barrier()
    pltpu.sync_copy(acc.at[pl.ds(s * _RPT, _RPT)],
                    out_hbm.at[c, pl.ds(s * _RPT, _RPT)])

  return sc_degree


_sc_degree = _make_sc_degree()


def _tc0_body(x_ref, degp_ref, mask_ref, w_ref, y_ref, dis_ref):
  deg = degp_ref[0] + degp_ref[1]
  dis = mask_ref[...] * lax.rsqrt(jnp.maximum(deg, 1.0))
  dis_ref[...] = dis
  y_ref[...] = dis * jnp.dot(x_ref[...], w_ref[...], preferred_element_type=_f32)


def _tc0(x_pad, degp, mask128, w0):
  return pl.pallas_call(
      _tc0_body,
      grid=(_GRID,),
      in_specs=[
          pl.BlockSpec((_BN, _F), lambda i: (i, 0)),
          pl.BlockSpec((_NC, _BN, _F), lambda i: (0, i, 0)),
          pl.BlockSpec((_BN, _F), lambda i: (i, 0)),
          pl.BlockSpec((_F, _F), lambda i: (0, 0)),
      ],
      out_specs=[
          pl.BlockSpec((_BN, _F), lambda i: (i, 0)),
          pl.BlockSpec((_BN, _F), lambda i: (i, 0)),
      ],
      out_shape=[
          jax.ShapeDtypeStruct((_NP, _F), _f32),
          jax.ShapeDtypeStruct((_NP, _F), _f32),
      ],
  )(x_pad, degp, mask128, w0)


def _tc_layer_body(dout, p_ref, yin_ref, dis_ref, b_ref, w_ref, out_ref, y_ref):
  dis = dis_ref[...]
  out = dis * (p_ref[0] + p_ref[1] + yin_ref[...]) + b_ref[...]
  out_ref[...] = out
  yw = jnp.dot(out, w_ref[...], preferred_element_type=_f32)
  y_ref[...] = dis[:, :dout] * yw


def _tc_layer(p, yin, disf, b_row, w, dout):
  return pl.pallas_call(
      functools.partial(_tc_layer_body, dout),
      grid=(_GRID,),
      in_specs=[
          pl.BlockSpec((_NC, _BN, _F), lambda i: (0, i, 0)),
          pl.BlockSpec((_BN, _F), lambda i: (i, 0)),
          pl.BlockSpec((_BN, _F), lambda i: (i, 0)),
          pl.BlockSpec((1, _F), lambda i: (0, 0)),
          pl.BlockSpec((_F, dout), lambda i: (0, 0)),
      ],
      out_specs=[
          pl.BlockSpec((_BN, _F), lambda i: (i, 0)),
          pl.BlockSpec((_BN, dout), lambda i: (i, 0)),
      ],
      out_shape=[
          jax.ShapeDtypeStruct((_NP, _F), _f32),
          jax.ShapeDtypeStruct((_NP, dout), _f32),
      ],
  )(p, yin, disf, b_row, w)


def _tc_final_body(p3_ref, y3_ref, dis_ref, b3_ref, o1_ref, o2_ref, o3_ref,
                   wa_ref, wb_ref, wc_ref, w4_ref, e_ref, bc_ref, res_ref):
  s3 = (p3_ref[0] + p3_ref[1] + y3_ref[...])[:, :16]
  out4_16 = dis_ref[...][:, :16] * s3 + b3_ref[...]
  out4_rep = jnp.dot(out4_16, e_ref[...], preferred_element_type=_f32)
  res = jnp.dot(o1_ref[...], wa_ref[...], preferred_element_type=_f32)
  res += jnp.dot(o2_ref[...], wb_ref[...], preferred_element_type=_f32)
  res += jnp.dot(o3_ref[...], wc_ref[...], preferred_element_type=_f32)
  res_ref[...] = res + out4_rep * w4_ref[...] + bc_ref[...]


def _tc_final(p3, y3, disf, b3_16, o1, o2, o3, wa, wb, wc, w4, e16x16, bc_row):
  return pl.pallas_call(
      _tc_final_body,
      grid=(_GRID,),
      in_specs=[
          pl.BlockSpec((_NC, _BN, _F), lambda i: (0, i, 0)),
          pl.BlockSpec((_BN, _F), lambda i: (i, 0)),
          pl.BlockSpec((_BN, _F), lambda i: (i, 0)),
          pl.BlockSpec((1, 16), lambda i: (0, 0)),
          pl.BlockSpec((_BN, _F), lambda i: (i, 0)),
          pl.BlockSpec((_BN, _F), lambda i: (i, 0)),
          pl.BlockSpec((_BN, _F), lambda i: (i, 0)),
          pl.BlockSpec((_F, 16), lambda i: (0, 0)),
          pl.BlockSpec((_F, 16), lambda i: (0, 0)),
          pl.BlockSpec((_F, 16), lambda i: (0, 0)),
          pl.BlockSpec((1, 16), lambda i: (0, 0)),
          pl.BlockSpec((16, 16), lambda i: (0, 0)),
          pl.BlockSpec((1, 16), lambda i: (0, 0)),
      ],
      out_specs=pl.BlockSpec((_BN, 16), lambda i: (i, 0)),
      out_shape=jax.ShapeDtypeStruct((_NP, 16), _f32),
  )(p3, y3, disf, b3_16, o1, o2, o3, wa, wb, wc, w4, e16x16, bc_row)


def kernel(x, edge_index, edge_attr, W0, b0, W1, b1, W2, b2, W3, b3, Wc, bc):
  src = edge_index[0].astype(jnp.int32)
  dst = edge_index[1].astype(jnp.int32)
  loops = jnp.arange(_N, dtype=jnp.int32)
  fill = jnp.full((_EALL - _E - _N,), _NP - 1, dtype=jnp.int32)
  dst_r = jnp.concatenate([dst, loops, fill]).reshape(_NW, _G // _GB, _GB, _CH)
  fill2 = _N + (jnp.arange(_EALL2 - _E, dtype=jnp.int32) % (_NP - _N))
  src_r2 = jnp.concatenate([src, fill2]).reshape(_NW, _G2 // _GB2, _GB2, _CH)
  dst_r2 = jnp.concatenate([dst, fill2]).reshape(_NW, _G2 // _GB2, _GB2, _CH)

  x_pad = jnp.pad(x, ((0, _NP - _N), (0, 0)))
  zeros128 = jnp.zeros((_RPT, _F), _f32)
  ones128 = jnp.ones((_CH, _F), _f32)
  mask128 = jnp.broadcast_to(
      (jnp.arange(_NP) < _N).astype(_f32)[:, None], (_NP, _F))
  e16x16 = jnp.zeros((16, 16), _f32).at[0].set(1.0)

  w3p = jnp.pad(W3, ((0, 0), (0, _F - 1)))
  wc_flat = Wc[:, 0, :]
  wa = wc_flat[:, 0:128].T
  wb = wc_flat[:, 128:256].T
  wcc = wc_flat[:, 256:384].T
  w4 = wc_flat[:, 384][None, :]
  b3_16 = jnp.broadcast_to(b3, (1, 16))
  bc_row = bc[None, :]

  degp = _sc_degree(ones128, dst_r, zeros128)
  y0, disf = _tc0(x_pad, degp, mask128, W0)
  p0 = _sc_scatter128(y0, src_r2, dst_r2, zeros128)
  out1, y1 = _tc_layer(p0, y0, disf, b0[None, :], W1, _F)
  p1 = _sc_scatter128(y1, src_r2, dst_r2, zeros128)
  out2, y2 = _tc_layer(p1, y1, disf, b1[None, :], W2, _F)
  p2 = _sc_scatter128(y2, src_r2, dst_r2, zeros128)
  out3, y3 = _tc_layer(p2, y2, disf, b2[None, :], w3p, _F)
  p3 = _sc_scatter128(y3, src_r2, dst_r2, zeros128)
  res = _tc_final(p3, y3, disf, b3_16, out1, out2, out3, wa, wb, wcc, w4,
                  e16x16, bc_row)
  return res[:_N]


# speedup vs baseline: 1.2538x; 1.1557x over previous
import functools

import jax
import jax.numpy as jnp
from jax import lax
from jax.experimental import pallas as pl
from jax.experimental.pallas import tpu as pltpu
from jax.experimental.pallas import tpu_sc as plsc

_N = 10000
_NP = 10240
_F = 128
_E = 320000
_NC = 2
_NS = 16
_NW = _NC * _NS
_CH = 128
_G = 81
_GB = 27
_EALL = _NW * _G * _CH
_G2 = 84
_GB2 = 28
_EALL2 = _NW * _G2 * _CH
_RPT = _NP // _NS
_BN = 512
_GRID = _NP // _BN

_f32 = jnp.float32


def _make_sc_scatter(depth):
  mesh = plsc.VectorSubcoreMesh(core_axis_name="c", subcore_axis_name="s")

  @functools.partial(
      pl.kernel,
      mesh=mesh,
      out_type=jax.ShapeDtypeStruct((_NC, _NP, depth), _f32),
      scratch_types=[
          pltpu.VMEM((_GB2, _CH), jnp.int32),
          pltpu.VMEM((_GB2, _CH), jnp.int32),
          pltpu.VMEM((2, _CH, depth), _f32),
          pltpu.VMEM_SHARED((_NP, depth), _f32),
          pltpu.SemaphoreType.DMA,
          pltpu.SemaphoreType.DMA,
      ],
  )
  def sc_scatter(y_hbm, src_hbm, dst_hbm, zeros_hbm, out_hbm,
                 idx_s, idx_d, rows, acc, sem, sem_s):
    c = lax.axis_index("c")
    s = lax.axis_index("s")
    pltpu.sync_copy(zeros_hbm, acc.at[pl.ds(s * _RPT, _RPT)])
    plsc.subcore_barrier()

    wid = s * _NC + c
    h = _CH // 2

    def fire_gather(j, b):
      pltpu.async_copy(y_hbm.at[idx_s.at[j, pl.ds(0, h)]],
                       rows.at[b, pl.ds(0, h)], sem)
      pltpu.async_copy(y_hbm.at[idx_s.at[j, pl.ds(h, h)]],
                       rows.at[b, pl.ds(h, h)], sem)

    def drain_gather(b):
      pltpu.make_async_copy(y_hbm.at[pl.ds(0, h)], rows.at[b, pl.ds(0, h)],
                            sem).wait()
      pltpu.make_async_copy(y_hbm.at[pl.ds(0, h)], rows.at[b, pl.ds(h, h)],
                            sem).wait()

    def run_blocks(b0, nb):
      def block(bi, carry):
        pltpu.sync_copy(src_hbm.at[wid, b0 + bi], idx_s)
        pltpu.sync_copy(dst_hbm.at[wid, b0 + bi], idx_d)
        fire_gather(0, 0)

        def body(j, carry2):
          drain_gather(j % 2)

          @pl.when(j >= 1)
          def _():
            pltpu.make_async_copy(y_hbm.at[pl.ds(0, _CH)],
                                  rows.at[(j + 1) % 2], sem_s).wait()

          @pl.when(j + 1 < _GB2)
          def _():
            fire_gather(j + 1, (j + 1) % 2)

          pltpu.async_copy(rows.at[j % 2], acc.at[idx_d.at[j]], sem_s,
                           add=True)
          return carry2

        r = lax.fori_loop(0, _GB2, body, carry)
        pltpu.make_async_copy(y_hbm.at[pl.ds(0, _CH)], rows.at[0],
                              sem_s).wait()
        return r

      lax.fori_loop(0, nb, block, 0)

    run_blocks(0, _G2 // _GB2)
    plsc.subcore_barrier()
    pltpu.sync_copy(acc.at[pl.ds(s * _RPT, _RPT)],
                    out_hbm.at[c, pl.ds(s * _RPT, _RPT)])

  return sc_scatter


_sc_scatter128 = _make_sc_scatter(_F)


def _make_sc_degree():
  mesh = plsc.VectorSubcoreMesh(core_axis_name="c", subcore_axis_name="s")

  @functools.partial(
      pl.kernel,
      mesh=mesh,
      out_type=jax.ShapeDtypeStruct((_NC, _NP, _F), _f32),
      scratch_types=[
          pltpu.VMEM((_GB, _CH), jnp.int32),
          pltpu.VMEM((_CH, _F), _f32),
          pltpu.VMEM_SHARED((_NP, _F), _f32),
          pltpu.SemaphoreType.DMA,
      ],
  )
  def sc_degree(ones_hbm, dst_hbm, zeros_hbm, out_hbm, idx_d, rows, acc, sem):
    c = lax.axis_index("c")
    s = lax.axis_index("s")
    wid = s * _NC + c
    pltpu.sync_copy(zeros_hbm, acc.at[pl.ds(s * _RPT, _RPT)])
    pltpu.sync_copy(ones_hbm, rows)
    plsc.subcore_barrier()

    def block(bi, carry):
      pltpu.sync_copy(dst_hbm.at[wid, bi], idx_d)

      def body(j, carry2):
        pltpu.sync_copy(rows, acc.at[idx_d.at[j]], add=True)
        return carry2

      return lax.fori_loop(0, _GB, body, carry)

    lax.fori_loop(0, _G // _GB, block, 0)
    plsc.subcore_barrier()
    pltpu.sync_copy(acc.at[pl.ds(s * _RPT, _RPT)],
                    out_hbm.at[c, pl.ds(s * _RPT, _RPT)])

  return sc_degree


_sc_degree = _make_sc_degree()


def _tc0_body(x_ref, degp_ref, mask_ref, w_ref, y_ref, dis_ref):
  deg = degp_ref[0] + degp_ref[1]
  dis = mask_ref[...] * lax.rsqrt(jnp.maximum(deg, 1.0))
  dis_ref[...] = dis
  y_ref[...] = dis * jnp.dot(x_ref[...], w_ref[...], preferred_element_type=_f32)


def _tc0(x_pad, degp, mask128, w0):
  return pl.pallas_call(
      _tc0_body,
      grid=(_GRID,),
      in_specs=[
          pl.BlockSpec((_BN, _F), lambda i: (i, 0)),
          pl.BlockSpec((_NC, _BN, _F), lambda i: (0, i, 0)),
          pl.BlockSpec((_BN, _F), lambda i: (i, 0)),
          pl.BlockSpec((_F, _F), lambda i: (0, 0)),
      ],
      out_specs=[
          pl.BlockSpec((_BN, _F), lambda i: (i, 0)),
          pl.BlockSpec((_BN, _F), lambda i: (i, 0)),
      ],
      out_shape=[
          jax.ShapeDtypeStruct((_NP, _F), _f32),
          jax.ShapeDtypeStruct((_NP, _F), _f32),
      ],
  )(x_pad, degp, mask128, w0)


def _tc_layer_body(dout, p_ref, yin_ref, dis_ref, b_ref, w_ref, out_ref, y_ref):
  dis = dis_ref[...]
  out = dis * (p_ref[0] + p_ref[1] + yin_ref[...]) + b_ref[...]
  out_ref[...] = out
  yw = jnp.dot(out, w_ref[...], preferred_element_type=_f32)
  y_ref[...] = dis[:, :dout] * yw


def _tc_layer(p, yin, disf, b_row, w, dout):
  return pl.pallas_call(
      functools.partial(_tc_layer_body, dout),
      grid=(_GRID,),
      in_specs=[
          pl.BlockSpec((_NC, _BN, _F), lambda i: (0, i, 0)),
          pl.BlockSpec((_BN, _F), lambda i: (i, 0)),
          pl.BlockSpec((_BN, _F), lambda i: (i, 0)),
          pl.BlockSpec((1, _F), lambda i: (0, 0)),
          pl.BlockSpec((_F, dout), lambda i: (0, 0)),
      ],
      out_specs=[
          pl.BlockSpec((_BN, _F), lambda i: (i, 0)),
          pl.BlockSpec((_BN, dout), lambda i: (i, 0)),
      ],
      out_shape=[
          jax.ShapeDtypeStruct((_NP, _F), _f32),
          jax.ShapeDtypeStruct((_NP, dout), _f32),
      ],
  )(p, yin, disf, b_row, w)


def _tc_final_body(p3_ref, y3_ref, dis_ref, b3_ref, o1_ref, o2_ref, o3_ref,
                   wa_ref, wb_ref, wc_ref, w4_ref, e_ref, bc_ref, res_ref):
  s3 = (p3_ref[0] + p3_ref[1] + y3_ref[...])[:, :16]
  out4_16 = dis_ref[...][:, :16] * s3 + b3_ref[...]
  out4_rep = jnp.dot(out4_16, e_ref[...], preferred_element_type=_f32)
  res = jnp.dot(o1_ref[...], wa_ref[...], preferred_element_type=_f32)
  res += jnp.dot(o2_ref[...], wb_ref[...], preferred_element_type=_f32)
  res += jnp.dot(o3_ref[...], wc_ref[...], preferred_element_type=_f32)
  res_ref[...] = res + out4_rep * w4_ref[...] + bc_ref[...]


def _tc_final(p3, y3, disf, b3_16, o1, o2, o3, wa, wb, wc, w4, e16x16, bc_row):
  return pl.pallas_call(
      _tc_final_body,
      grid=(_GRID,),
      in_specs=[
          pl.BlockSpec((_NC, _BN, _F), lambda i: (0, i, 0)),
          pl.BlockSpec((_BN, _F), lambda i: (i, 0)),
          pl.BlockSpec((_BN, _F), lambda i: (i, 0)),
          pl.BlockSpec((1, 16), lambda i: (0, 0)),
          pl.BlockSpec((_BN, _F), lambda i: (i, 0)),
          pl.BlockSpec((_BN, _F), lambda i: (i, 0)),
          pl.BlockSpec((_BN, _F), lambda i: (i, 0)),
          pl.BlockSpec((_F, 16), lambda i: (0, 0)),
          pl.BlockSpec((_F, 16), lambda i: (0, 0)),
          pl.BlockSpec((_F, 16), lambda i: (0, 0)),
          pl.BlockSpec((1, 16), lambda i: (0, 0)),
          pl.BlockSpec((16, 16), lambda i: (0, 0)),
          pl.BlockSpec((1, 16), lambda i: (0, 0)),
      ],
      out_specs=pl.BlockSpec((_BN, 16), lambda i: (i, 0)),
      out_shape=jax.ShapeDtypeStruct((_NP, 16), _f32),
  )(p3, y3, disf, b3_16, o1, o2, o3, wa, wb, wc, w4, e16x16, bc_row)


def kernel(x, edge_index, edge_attr, W0, b0, W1, b1, W2, b2, W3, b3, Wc, bc):
  src = edge_index[0].astype(jnp.int32)
  dst = edge_index[1].astype(jnp.int32)
  loops = jnp.arange(_N, dtype=jnp.int32)
  fill = jnp.full((_EALL - _E - _N,), _NP - 1, dtype=jnp.int32)
  dst_r = jnp.concatenate([dst, loops, fill]).reshape(_NW, _G // _GB, _GB, _CH)
  fill2 = _N + (jnp.arange(_EALL2 - _E, dtype=jnp.int32) % (_NP - _N))
  src_r2 = jnp.concatenate([src, fill2]).reshape(_NW, _G2 // _GB2, _GB2, _CH)
  dst_r2 = jnp.concatenate([dst, fill2]).reshape(_NW, _G2 // _GB2, _GB2, _CH)

  x_pad = jnp.pad(x, ((0, _NP - _N), (0, 0)))
  zeros128 = jnp.zeros((_RPT, _F), _f32)
  ones128 = jnp.ones((_CH, _F), _f32)
  mask128 = jnp.broadcast_to(
      (jnp.arange(_NP) < _N).astype(_f32)[:, None], (_NP, _F))
  e16x16 = jnp.zeros((16, 16), _f32).at[0].set(1.0)

  w3p = jnp.pad(W3, ((0, 0), (0, _F - 1)))
  wc_flat = Wc[:, 0, :]
  wa = wc_flat[:, 0:128].T
  wb = wc_flat[:, 128:256].T
  wcc = wc_flat[:, 256:384].T
  w4 = wc_flat[:, 384][None, :]
  b3_16 = jnp.broadcast_to(b3, (1, 16))
  bc_row = bc[None, :]

  degp = _sc_degree(ones128, dst_r, zeros128)
  y0, disf = _tc0(x_pad, degp, mask128, W0)
  p0 = _sc_scatter128(y0, src_r2, dst_r2, zeros128)
  out1, y1 = _tc_layer(p0, y0, disf, b0[None, :], W1, _F)
  p1 = _sc_scatter128(y1, src_r2, dst_r2, zeros128)
  out2, y2 = _tc_layer(p1, y1, disf, b1[None, :], W2, _F)
  p2 = _sc_scatter128(y2, src_r2, dst_r2, zeros128)
  out3, y3 = _tc_layer(p2, y2, disf, b2[None, :], w3p, _F)
  p3 = _sc_scatter128(y3, src_r2, dst_r2, zeros128)
  res = _tc_final(p3, y3, disf, b3_16, out1, out2, out3, wa, wb, wcc, w4,
                  e16x16, bc_row)
  return res[:_N]
